# baseline (speedup 1.0000x reference)
import jax
import jax.numpy as jnp
from jax.experimental import pallas as pl

_N_GRID = 128
_R_S = 48
_R_C = 144
_P = 27
_CH = 128
_SIGMA_BIAS = -5.0
_BLK = 1024
_NM = 66


def _leaky(x):
    return jnp.where(x >= 0, x, 0.01 * x)


def _sigmoid(x):
    z = jnp.exp(-jnp.abs(x))
    return jnp.where(x >= 0, 1.0 / (1.0 + z), z / (1.0 + z))


def _softplus(x):
    return jnp.maximum(x, 0.0) + jnp.log1p(jnp.exp(-jnp.abs(x)))


def _tc_body(xyz_ref, dirs_ref, voxel_ref, fl_ref, fr_ref, ml_ref, mr_ref,
             ones_ref, bpa_ref, bpb_ref, e_ref, asin_ref, acos_ref,
             w2_ref, w3_ref, b1_ref, b2_ref, b3_ref, sig_ref, rgb_ref):
    prod_f = None
    prod_m = None
    for a in range(3):
        xa = xyz_ref[:, a][:, None]
        vox = voxel_ref[a][None, :]
        cmp = (vox < xa).astype(jnp.float32)
        fl = jnp.dot(cmp, fl_ref[a], preferred_element_type=jnp.float32)
        fr = jnp.dot(cmp, fr_ref[a], preferred_element_type=jnp.float32)
        ml = jnp.dot(cmp, ml_ref[a], preferred_element_type=jnp.float32)
        mr = jnp.dot(cmp, mr_ref[a], preferred_element_type=jnp.float32)
        vl = ml[:, _NM - 2][:, None]
        vr = ml[:, _NM - 1][:, None]
        lerp = (xa - vl) / (vr - vl + 1e-06)
        gf = fl + lerp * (fr - fl)
        gm = ml + lerp * (mr - ml)
        prod_f = gf if prod_f is None else prod_f * gf
        prod_m = gm if prod_m is None else prod_m * gm

    sig_raw = jnp.dot(prod_m, ones_ref[...],
                      preferred_element_type=jnp.float32)[:, 0] + _SIGMA_BIAS
    sig_ref[...] = _softplus(sig_raw)

    t = (jnp.dot(prod_f, bpa_ref[...], preferred_element_type=jnp.float32)
         + jnp.dot(prod_m, bpb_ref[...], preferred_element_type=jnp.float32)
         + jnp.dot(dirs_ref[...], e_ref[...],
                   preferred_element_type=jnp.float32))
    pre = (jnp.dot(jnp.sin(t), asin_ref[...],
                   preferred_element_type=jnp.float32)
           + jnp.dot(jnp.cos(t), acos_ref[...],
                     preferred_element_type=jnp.float32)
           + b1_ref[...][None, :])
    h1 = _leaky(pre)
    h2 = _leaky(jnp.dot(h1, w2_ref[...],
                        preferred_element_type=jnp.float32) + b2_ref[...][None, :])
    rgb_ref[...] = _sigmoid(
        jnp.dot(h2, w3_ref[...], preferred_element_type=jnp.float32)
        + b3_ref[...][None, :])


def _ldiff(t):
    return jnp.concatenate([t[:, :1], t[:, 1:] - t[:, :-1]], axis=1)


def _rdiff(t):
    z = jnp.zeros_like(t[:, :1])
    return jnp.concatenate([t[:, 1:2], t[:, 2:] - t[:, 1:-1], z], axis=1)


@jax.jit
def kernel(xyz, directions, voxel, sigma, feature, B, W1, b1, W2, b2, W3, b3):
    npts = xyz.shape[0]
    grid = npts // _BLK

    tf1 = jnp.transpose(feature[:, :_CH, :], (0, 2, 1))
    tmx = jnp.concatenate(
        [jnp.transpose(sigma, (0, 2, 1)),
         jnp.transpose(feature[:, _CH:, :], (0, 2, 1))], axis=2)
    fl = _ldiff(tf1)
    fr = _rdiff(tf1)
    vcols = voxel[:, :, None]
    ml = jnp.concatenate([_ldiff(tmx), _ldiff(vcols), _rdiff(vcols)], axis=2)
    mr = jnp.concatenate([_rdiff(tmx), jnp.zeros((3, _N_GRID, 2), jnp.float32)],
                         axis=2)

    ones48 = jnp.zeros((_NM, 1), jnp.float32).at[:_R_S].set(1.0)

    bp2 = (jnp.zeros((_R_C, _CH), jnp.float32)
           .at[:, :_P].set(B).at[:, _P:2 * _P].set(2.0 * B))
    bpa = bp2[:_CH]
    bpb = jnp.zeros((_NM, _CH), jnp.float32).at[_R_S:64].set(bp2[_CH:])
    e = jnp.zeros((3, _CH), jnp.float32)
    for i in range(3):
        e = e.at[i, 54 + i].set(1.0).at[i, 57 + i].set(2.0)
    w1t = W1.T
    asin = (jnp.zeros((_CH, _CH), jnp.float32)
            .at[:_P].set(w1t[0:27]).at[_P:2 * _P].set(w1t[54:81])
            .at[54:57].set(w1t[108:111]).at[57:60].set(w1t[114:117]))
    acos = (jnp.zeros((_CH, _CH), jnp.float32)
            .at[:_P].set(w1t[27:54]).at[_P:2 * _P].set(w1t[81:108])
            .at[54:57].set(w1t[111:114]).at[57:60].set(w1t[117:120]))

    full = lambda *shape: pl.BlockSpec(shape, lambda i: (0,) * len(shape))
    sig, rgb = pl.pallas_call(
        _tc_body,
        grid=(grid,),
        in_specs=[
            pl.BlockSpec((_BLK, 3), lambda i: (i, 0)),
            pl.BlockSpec((_BLK, 3), lambda i: (i, 0)),
            full(3, _N_GRID),
            full(3, _N_GRID, _CH),
            full(3, _N_GRID, _CH),
            full(3, _N_GRID, _NM),
            full(3, _N_GRID, _NM),
            full(_NM, 1),
            full(_CH, _CH),
            full(_NM, _CH),
            full(3, _CH),
            full(_CH, _CH),
            full(_CH, _CH),
            full(_CH, _CH),
            full(_CH, 3),
            full(_CH),
            full(_CH),
            full(3),
        ],
        out_specs=[
            pl.BlockSpec((_BLK,), lambda i: (i,)),
            pl.BlockSpec((_BLK, 3), lambda i: (i, 0)),
        ],
        out_shape=[
            jax.ShapeDtypeStruct((npts,), jnp.float32),
            jax.ShapeDtypeStruct((npts, 3), jnp.float32),
        ],
    )(xyz, directions, voxel, fl, fr, ml, mr, ones48, bpa, bpb,
      e, asin, acos, W2.T, W3.T, b1, b2, b3)
    return sig, rgb


# speedup vs baseline: 1.1162x; 1.1162x over previous
import jax
import jax.numpy as jnp
from jax.experimental import pallas as pl

_N_GRID = 128
_R_S = 48
_R_C = 144
_P = 27
_CH = 128
_SIGMA_BIAS = -5.0
_BLK = 4096
_NM = 66


def _leaky(x):
    return jnp.where(x >= 0, x, 0.01 * x)


def _sigmoid(x):
    z = jnp.exp(-jnp.abs(x))
    return jnp.where(x >= 0, 1.0 / (1.0 + z), z / (1.0 + z))


def _softplus(x):
    return jnp.maximum(x, 0.0) + jnp.log1p(jnp.exp(-jnp.abs(x)))


def _tc_body(xyz_ref, dirs_ref, voxel_ref, fl_ref, fr_ref, ml_ref, mr_ref,
             ones_ref, bpa_ref, bpb_ref, e_ref, asin_ref, acos_ref,
             w2_ref, w3_ref, b1_ref, b2_ref, b3_ref, sig_ref, rgb_ref):
    prod_f = None
    prod_m = None
    for a in range(3):
        xa = xyz_ref[:, a][:, None]
        vox = voxel_ref[a][None, :]
        cmp = (vox < xa).astype(jnp.float32)
        fl = jnp.dot(cmp, fl_ref[a], preferred_element_type=jnp.float32)
        fr = jnp.dot(cmp, fr_ref[a], preferred_element_type=jnp.float32)
        ml = jnp.dot(cmp, ml_ref[a], preferred_element_type=jnp.float32)
        mr = jnp.dot(cmp, mr_ref[a], preferred_element_type=jnp.float32)
        vl = ml[:, _NM - 2][:, None]
        vr = ml[:, _NM - 1][:, None]
        lerp = (xa - vl) / (vr - vl + 1e-06)
        gf = fl + lerp * (fr - fl)
        gm = ml + lerp * (mr - ml)
        prod_f = gf if prod_f is None else prod_f * gf
        prod_m = gm if prod_m is None else prod_m * gm

    sig_raw = jnp.dot(prod_m, ones_ref[...],
                      preferred_element_type=jnp.float32)[:, 0] + _SIGMA_BIAS
    sig_ref[...] = _softplus(sig_raw)

    t = (jnp.dot(prod_f, bpa_ref[...], preferred_element_type=jnp.float32)
         + jnp.dot(prod_m, bpb_ref[...], preferred_element_type=jnp.float32)
         + jnp.dot(dirs_ref[...], e_ref[...],
                   preferred_element_type=jnp.float32))
    pre = (jnp.dot(jnp.sin(t), asin_ref[...],
                   preferred_element_type=jnp.float32)
           + jnp.dot(jnp.cos(t), acos_ref[...],
                     preferred_element_type=jnp.float32)
           + b1_ref[...][None, :])
    h1 = _leaky(pre)
    h2 = _leaky(jnp.dot(h1, w2_ref[...],
                        preferred_element_type=jnp.float32) + b2_ref[...][None, :])
    rgb_ref[...] = _sigmoid(
        jnp.dot(h2, w3_ref[...], preferred_element_type=jnp.float32)
        + b3_ref[...][None, :])


def _ldiff(t):
    return jnp.concatenate([t[:, :1], t[:, 1:] - t[:, :-1]], axis=1)


def _rdiff(t):
    z = jnp.zeros_like(t[:, :1])
    return jnp.concatenate([t[:, 1:2], t[:, 2:] - t[:, 1:-1], z], axis=1)


@jax.jit
def kernel(xyz, directions, voxel, sigma, feature, B, W1, b1, W2, b2, W3, b3):
    npts = xyz.shape[0]
    grid = npts // _BLK

    tf1 = jnp.transpose(feature[:, :_CH, :], (0, 2, 1))
    tmx = jnp.concatenate(
        [jnp.transpose(sigma, (0, 2, 1)),
         jnp.transpose(feature[:, _CH:, :], (0, 2, 1))], axis=2)
    fl = _ldiff(tf1)
    fr = _rdiff(tf1)
    vcols = voxel[:, :, None]
    ml = jnp.concatenate([_ldiff(tmx), _ldiff(vcols), _rdiff(vcols)], axis=2)
    mr = jnp.concatenate([_rdiff(tmx), jnp.zeros((3, _N_GRID, 2), jnp.float32)],
                         axis=2)

    ones48 = jnp.zeros((_NM, 1), jnp.float32).at[:_R_S].set(1.0)

    bp2 = (jnp.zeros((_R_C, _CH), jnp.float32)
           .at[:, :_P].set(B).at[:, _P:2 * _P].set(2.0 * B))
    bpa = bp2[:_CH]
    bpb = jnp.zeros((_NM, _CH), jnp.float32).at[_R_S:64].set(bp2[_CH:])
    e = jnp.zeros((3, _CH), jnp.float32)
    for i in range(3):
        e = e.at[i, 54 + i].set(1.0).at[i, 57 + i].set(2.0)
    w1t = W1.T
    asin = (jnp.zeros((_CH, _CH), jnp.float32)
            .at[:_P].set(w1t[0:27]).at[_P:2 * _P].set(w1t[54:81])
            .at[54:57].set(w1t[108:111]).at[57:60].set(w1t[114:117]))
    acos = (jnp.zeros((_CH, _CH), jnp.float32)
            .at[:_P].set(w1t[27:54]).at[_P:2 * _P].set(w1t[81:108])
            .at[54:57].set(w1t[111:114]).at[57:60].set(w1t[117:120]))

    full = lambda *shape: pl.BlockSpec(shape, lambda i: (0,) * len(shape))
    sig, rgb = pl.pallas_call(
        _tc_body,
        grid=(grid,),
        in_specs=[
            pl.BlockSpec((_BLK, 3), lambda i: (i, 0)),
            pl.BlockSpec((_BLK, 3), lambda i: (i, 0)),
            full(3, _N_GRID),
            full(3, _N_GRID, _CH),
            full(3, _N_GRID, _CH),
            full(3, _N_GRID, _NM),
            full(3, _N_GRID, _NM),
            full(_NM, 1),
            full(_CH, _CH),
            full(_NM, _CH),
            full(3, _CH),
            full(_CH, _CH),
            full(_CH, _CH),
            full(_CH, _CH),
            full(_CH, 3),
            full(_CH),
            full(_CH),
            full(3),
        ],
        out_specs=[
            pl.BlockSpec((_BLK,), lambda i: (i,)),
            pl.BlockSpec((_BLK, 3), lambda i: (i, 0)),
        ],
        out_shape=[
            jax.ShapeDtypeStruct((npts,), jnp.float32),
            jax.ShapeDtypeStruct((npts, 3), jnp.float32),
        ],
    )(xyz, directions, voxel, fl, fr, ml, mr, ones48, bpa, bpb,
      e, asin, acos, W2.T, W3.T, b1, b2, b3)
    return sig, rgb


# speedup vs baseline: 2.6363x; 2.3619x over previous
import jax
import jax.numpy as jnp
from jax.experimental import pallas as pl

_N_GRID = 128
_R_S = 48
_P = 27
_CH = 128
_SIGMA_BIAS = -5.0
_BLK = 4096
_NM = 66
_NT = 64


def _leaky(x):
    return jnp.where(x >= 0, x, 0.01 * x)


def _sigmoid(x):
    z = jnp.exp(-jnp.abs(x))
    return jnp.where(x >= 0, 1.0 / (1.0 + z), z / (1.0 + z))


def _softplus(x):
    return jnp.maximum(x, 0.0) + jnp.log1p(jnp.exp(-jnp.abs(x)))


def _tc_body(xyz_ref, dirs_ref, voxel_ref, fl_ref, fr_ref, ml_ref, mr_ref,
             ones_ref, bpa_ref, bpb_ref, e_ref, asin_ref, acos_ref,
             w2_ref, w3_ref, b1_ref, b2_ref, b3_ref, sig_ref, rgb_ref):
    prod_f = None
    prod_m = None
    for a in range(3):
        xa = xyz_ref[a][None, :]
        vox = voxel_ref[:, a][:, None]
        cmp = (vox < xa).astype(jnp.float32)
        fl = jnp.dot(fl_ref[a], cmp, preferred_element_type=jnp.float32)
        fr = jnp.dot(fr_ref[a], cmp, preferred_element_type=jnp.float32)
        ml = jnp.dot(ml_ref[a], cmp, preferred_element_type=jnp.float32)
        mr = jnp.dot(mr_ref[a], cmp, preferred_element_type=jnp.float32)
        vl = ml[_NM - 2][None, :]
        vr = ml[_NM - 1][None, :]
        lerp = (xa - vl) / (vr - vl + 1e-06)
        gf = fl + lerp * (fr - fl)
        gm = ml + lerp * (mr - ml)
        prod_f = gf if prod_f is None else prod_f * gf
        prod_m = gm if prod_m is None else prod_m * gm

    sig_raw = jnp.dot(ones_ref[...], prod_m,
                      preferred_element_type=jnp.float32) + _SIGMA_BIAS
    sig_ref[...] = _softplus(sig_raw)

    t = (jnp.dot(bpa_ref[...], prod_f, preferred_element_type=jnp.float32)
         + jnp.dot(bpb_ref[...], prod_m, preferred_element_type=jnp.float32)
         + jnp.dot(e_ref[...], dirs_ref[...],
                   preferred_element_type=jnp.float32))
    pre = (jnp.dot(asin_ref[...], jnp.sin(t),
                   preferred_element_type=jnp.float32)
           + jnp.dot(acos_ref[...], jnp.cos(t),
                     preferred_element_type=jnp.float32)
           + b1_ref[...])
    h1 = _leaky(pre)
    h2 = _leaky(jnp.dot(w2_ref[...], h1,
                        preferred_element_type=jnp.float32) + b2_ref[...])
    rgb_ref[...] = _sigmoid(
        jnp.dot(w3_ref[...], h2, preferred_element_type=jnp.float32)
        + b3_ref[...])


def _ldiff(t):
    return jnp.concatenate([t[..., :1], t[..., 1:] - t[..., :-1]], axis=-1)


def _rdiff(t):
    z = jnp.zeros_like(t[..., :1])
    return jnp.concatenate([t[..., 1:2], t[..., 2:] - t[..., 1:-1], z], axis=-1)


@jax.jit
def kernel(xyz, directions, voxel, sigma, feature, B, W1, b1, W2, b2, W3, b3):
    npts = xyz.shape[0]
    grid = npts // _BLK

    xyz_t = xyz.T
    dirs_t = directions.T
    vox_t = voxel.T

    f1 = feature[:, :_CH, :]
    tmx = jnp.concatenate(
        [sigma, feature[:, _CH:, :],
         jnp.broadcast_to(voxel[:, None, :], (3, 1, _N_GRID)),
         jnp.broadcast_to(voxel[:, None, :], (3, 1, _N_GRID))],
        axis=1)
    fl = _ldiff(f1)
    fr = _rdiff(f1)
    ml = jnp.concatenate([_ldiff(tmx[:, :_NM - 1]), _rdiff(tmx[:, _NM - 1:])],
                         axis=1)
    mr = jnp.concatenate([_rdiff(tmx[:, :_NM - 2]),
                          jnp.zeros((3, 2, _N_GRID), jnp.float32)], axis=1)

    ones48 = jnp.zeros((1, _NM), jnp.float32).at[0, :_R_S].set(1.0)

    bt = B.T
    bpa = (jnp.zeros((_NT, _CH), jnp.float32)
           .at[:_P].set(bt[:, :_CH]).at[_P:2 * _P].set(2.0 * bt[:, :_CH]))
    bpb = (jnp.zeros((_NT, _NM), jnp.float32)
           .at[:_P, _R_S:64].set(bt[:, _CH:])
           .at[_P:2 * _P, _R_S:64].set(2.0 * bt[:, _CH:]))
    e = jnp.zeros((_NT, 3), jnp.float32)
    for i in range(3):
        e = e.at[54 + i, i].set(1.0).at[57 + i, i].set(2.0)
    asin = (jnp.zeros((_CH, _NT), jnp.float32)
            .at[:, :_P].set(W1[:, 0:27]).at[:, _P:2 * _P].set(W1[:, 54:81])
            .at[:, 54:57].set(W1[:, 108:111]).at[:, 57:60].set(W1[:, 114:117]))
    acos = (jnp.zeros((_CH, _NT), jnp.float32)
            .at[:, :_P].set(W1[:, 27:54]).at[:, _P:2 * _P].set(W1[:, 81:108])
            .at[:, 54:57].set(W1[:, 111:114]).at[:, 57:60].set(W1[:, 117:120]))

    full = lambda *shape: pl.BlockSpec(shape, lambda i: (0,) * len(shape))
    sig, rgb = pl.pallas_call(
        _tc_body,
        grid=(grid,),
        in_specs=[
            pl.BlockSpec((3, _BLK), lambda i: (0, i)),
            pl.BlockSpec((3, _BLK), lambda i: (0, i)),
            full(_N_GRID, 3),
            full(3, _CH, _N_GRID),
            full(3, _CH, _N_GRID),
            full(3, _NM, _N_GRID),
            full(3, _NM, _N_GRID),
            full(1, _NM),
            full(_NT, _CH),
            full(_NT, _NM),
            full(_NT, 3),
            full(_CH, _NT),
            full(_CH, _NT),
            full(_CH, _CH),
            full(3, _CH),
            full(_CH, 1),
            full(_CH, 1),
            full(3, 1),
        ],
        out_specs=[
            pl.BlockSpec((1, _BLK), lambda i: (0, i)),
            pl.BlockSpec((3, _BLK), lambda i: (0, i)),
        ],
        out_shape=[
            jax.ShapeDtypeStruct((1, npts), jnp.float32),
            jax.ShapeDtypeStruct((3, npts), jnp.float32),
        ],
    )(xyz_t, dirs_t, vox_t, fl, fr, ml, mr, ones48, bpa, bpb,
      e, asin, acos, W2, W3, b1[:, None], b2[:, None], b3[:, None])
    return sig[0], rgb.T


# speedup vs baseline: 2.7443x; 1.0410x over previous
import jax
import jax.numpy as jnp
from jax.experimental import pallas as pl

_N_GRID = 128
_R_S = 48
_P = 27
_CH = 128
_SIGMA_BIAS = -5.0
_BLK = 4096
_NS = 392


def _leaky(x):
    return jnp.where(x >= 0, x, 0.01 * x)


def _sigmoid(x):
    z = jnp.exp(-jnp.abs(x))
    return jnp.where(x >= 0, 1.0 / (1.0 + z), z / (1.0 + z))


def _softplus(x):
    return jnp.maximum(x, 0.0) + jnp.log1p(jnp.exp(-jnp.abs(x)))


def _tc_body(xyz_ref, dirs_ref, voxel_ref, ts_ref, bpa_ref, bpb_ref,
             asin_ref, acos_ref, w2_ref, w3_ref, b1_ref, b2_ref, b3_ref,
             sig_ref, rgb_ref):
    blk = xyz_ref.shape[1]
    prod_f = None
    prod_m = None
    for a in range(3):
        xa = xyz_ref[a][None, :]
        vox = voxel_ref[:, a][:, None]
        cmp = (vox < xa).astype(jnp.float32)
        g = jnp.dot(ts_ref[a], cmp, preferred_element_type=jnp.float32)
        vl = g[384:385]
        vr = g[385:386]
        lerp = (xa - vl) / (vr - vl + 1e-06)
        gf = g[0:128] + lerp * (g[128:256] - g[0:128])
        gm = g[256:320] + lerp * (g[320:384] - g[256:320])
        prod_f = gf if prod_f is None else prod_f * gf
        prod_m = gm if prod_m is None else prod_m * gm

    t2 = (jnp.dot(bpa_ref[...], prod_f, preferred_element_type=jnp.float32)
          + jnp.dot(bpb_ref[...], prod_m, preferred_element_type=jnp.float32))
    sig_ref[...] = _softplus(t2[56:57] + _SIGMA_BIAS)

    d = dirs_ref[...]
    td = jnp.concatenate([d, d + d, jnp.zeros((2, blk), jnp.float32)], axis=0)
    t = jnp.concatenate([t2[0:56], td], axis=0)
    pre = (jnp.dot(asin_ref[...], jnp.sin(t),
                   preferred_element_type=jnp.float32)
           + jnp.dot(acos_ref[...], jnp.cos(t),
                     preferred_element_type=jnp.float32)
           + b1_ref[...])
    h1 = _leaky(pre)
    h2 = _leaky(jnp.dot(w2_ref[...], h1,
                        preferred_element_type=jnp.float32) + b2_ref[...])
    rgb_ref[...] = _sigmoid(
        jnp.dot(w3_ref[...], h2, preferred_element_type=jnp.float32)
        + b3_ref[...])


def _ldiff(t):
    return jnp.concatenate([t[..., :1], t[..., 1:] - t[..., :-1]], axis=-1)


def _rdiff(t):
    z = jnp.zeros_like(t[..., :1])
    return jnp.concatenate([t[..., 1:2], t[..., 2:] - t[..., 1:-1], z], axis=-1)


@jax.jit
def kernel(xyz, directions, voxel, sigma, feature, B, W1, b1, W2, b2, W3, b3):
    npts = xyz.shape[0]
    grid = npts // _BLK

    xyz_t = xyz.T
    dirs_t = directions.T
    vox_t = voxel.T

    f1 = feature[:, :_CH, :]
    mix = jnp.concatenate([sigma, feature[:, _CH:, :]], axis=1)
    vrow = voxel[:, None, :]
    ts = jnp.concatenate(
        [_ldiff(f1), _rdiff(f1), _ldiff(mix), _rdiff(mix),
         _ldiff(vrow), _rdiff(vrow),
         jnp.zeros((3, _NS - 386, _N_GRID), jnp.float32)], axis=1)

    bt = B.T
    bpa = (jnp.zeros((_CH // 2, _CH), jnp.float32)
           .at[:_P].set(bt[:, :_CH]).at[_P:2 * _P].set(2.0 * bt[:, :_CH]))
    bpb = (jnp.zeros((_CH // 2, _CH // 2), jnp.float32)
           .at[:_P, _R_S:].set(bt[:, _CH:])
           .at[_P:2 * _P, _R_S:].set(2.0 * bt[:, _CH:])
           .at[56, :_R_S].set(1.0))
    asin = (jnp.zeros((_CH, _CH // 2), jnp.float32)
            .at[:, :_P].set(W1[:, 0:27]).at[:, _P:2 * _P].set(W1[:, 54:81])
            .at[:, 56:59].set(W1[:, 108:111]).at[:, 59:62].set(W1[:, 114:117]))
    acos = (jnp.zeros((_CH, _CH // 2), jnp.float32)
            .at[:, :_P].set(W1[:, 27:54]).at[:, _P:2 * _P].set(W1[:, 81:108])
            .at[:, 56:59].set(W1[:, 111:114]).at[:, 59:62].set(W1[:, 117:120]))

    full = lambda *shape: pl.BlockSpec(shape, lambda i: (0,) * len(shape))
    sig, rgb = pl.pallas_call(
        _tc_body,
        grid=(grid,),
        in_specs=[
            pl.BlockSpec((3, _BLK), lambda i: (0, i)),
            pl.BlockSpec((3, _BLK), lambda i: (0, i)),
            full(_N_GRID, 3),
            full(3, _NS, _N_GRID),
            full(_CH // 2, _CH),
            full(_CH // 2, _CH // 2),
            full(_CH, _CH // 2),
            full(_CH, _CH // 2),
            full(_CH, _CH),
            full(3, _CH),
            full(_CH, 1),
            full(_CH, 1),
            full(3, 1),
        ],
        out_specs=[
            pl.BlockSpec((1, _BLK), lambda i: (0, i)),
            pl.BlockSpec((3, _BLK), lambda i: (0, i)),
        ],
        out_shape=[
            jax.ShapeDtypeStruct((1, npts), jnp.float32),
            jax.ShapeDtypeStruct((3, npts), jnp.float32),
        ],
    )(xyz_t, dirs_t, vox_t, ts, bpa, bpb, asin, acos,
      W2, W3, b1[:, None], b2[:, None], b3[:, None])
    return sig[0], rgb.T


# speedup vs baseline: 2.7937x; 1.0180x over previous
import jax
import jax.numpy as jnp
from jax.experimental import pallas as pl

_N_GRID = 128
_R_S = 48
_P = 27
_CH = 128
_SIGMA_BIAS = -5.0
_BLK = 4096
_NR = 192


def _leaky(x):
    return jnp.where(x >= 0, x, 0.01 * x)


def _sigmoid(x):
    z = jnp.exp(-jnp.abs(x))
    return jnp.where(x >= 0, 1.0 / (1.0 + z), z / (1.0 + z))


def _softplus(x):
    return jnp.maximum(x, 0.0) + jnp.log1p(jnp.exp(-jnp.abs(x)))


def _tc_body(xyz_ref, dirs_ref, voxel_ref, tl_ref, td_ref, zv_ref,
             bp_ref, asin_ref, acos_ref, w2_ref, w3_ref,
             b1_ref, b2_ref, b3_ref, sig_ref, rgb_ref):
    blk = xyz_ref.shape[1]
    prod = None
    for a in range(3):
        xa = xyz_ref[a][None, :]
        vox = voxel_ref[:, a][:, None]
        c = vox < xa
        cmp = jnp.where(c, 1.0, 0.0)
        vlr = jnp.dot(zv_ref[a], cmp, preferred_element_type=jnp.float32)
        vl = vlr[0:1]
        vr = vlr[1:2]
        lerp = (xa - vl) / (vr - vl + 1e-06)
        cmpl = jnp.where(c, jnp.broadcast_to(lerp, c.shape), 0.0)
        ga = (jnp.dot(tl_ref[a], cmp, preferred_element_type=jnp.float32)
              + jnp.dot(td_ref[a], cmpl, preferred_element_type=jnp.float32))
        prod = ga if prod is None else prod * ga

    t2 = jnp.dot(bp_ref[...], prod, preferred_element_type=jnp.float32)
    sig_ref[...] = _softplus(t2[56:57] + _SIGMA_BIAS)

    d = dirs_ref[...]
    td = jnp.concatenate([d, d + d, jnp.zeros((2, blk), jnp.float32)], axis=0)
    t = jnp.concatenate([t2[0:56], td], axis=0)
    pre = (jnp.dot(asin_ref[...], jnp.sin(t),
                   preferred_element_type=jnp.float32)
           + jnp.dot(acos_ref[...], jnp.cos(t),
                     preferred_element_type=jnp.float32)
           + b1_ref[...])
    h1 = _leaky(pre)
    h2 = _leaky(jnp.dot(w2_ref[...], h1,
                        preferred_element_type=jnp.float32) + b2_ref[...])
    rgb_ref[...] = _sigmoid(
        jnp.dot(w3_ref[...], h2, preferred_element_type=jnp.float32)
        + b3_ref[...])


def _ldiff(t):
    return jnp.concatenate([t[..., :1], t[..., 1:] - t[..., :-1]], axis=-1)


def _rdiff(t):
    z = jnp.zeros_like(t[..., :1])
    return jnp.concatenate([t[..., 1:2], t[..., 2:] - t[..., 1:-1], z], axis=-1)


@jax.jit
def kernel(xyz, directions, voxel, sigma, feature, B, W1, b1, W2, b2, W3, b3):
    npts = xyz.shape[0]
    grid = npts // _BLK

    xyz_t = xyz.T
    dirs_t = directions.T
    vox_t = voxel.T

    stack = jnp.concatenate([feature[:, :_CH, :], sigma, feature[:, _CH:, :]],
                            axis=1)
    tl = _ldiff(stack)
    td = _rdiff(stack) - tl
    vrow = voxel[:, None, :]
    zv = jnp.concatenate([_ldiff(vrow), _rdiff(vrow)], axis=1)

    bt = B.T
    bp = (jnp.zeros((_CH // 2, _NR), jnp.float32)
          .at[:_P, :_CH].set(bt[:, :_CH])
          .at[:_P, _CH + _R_S:].set(bt[:, _CH:])
          .at[_P:2 * _P, :_CH].set(2.0 * bt[:, :_CH])
          .at[_P:2 * _P, _CH + _R_S:].set(2.0 * bt[:, _CH:])
          .at[56, _CH:_CH + _R_S].set(1.0))
    asin = (jnp.zeros((_CH, _CH // 2), jnp.float32)
            .at[:, :_P].set(W1[:, 0:27]).at[:, _P:2 * _P].set(W1[:, 54:81])
            .at[:, 56:59].set(W1[:, 108:111]).at[:, 59:62].set(W1[:, 114:117]))
    acos = (jnp.zeros((_CH, _CH // 2), jnp.float32)
            .at[:, :_P].set(W1[:, 27:54]).at[:, _P:2 * _P].set(W1[:, 81:108])
            .at[:, 56:59].set(W1[:, 111:114]).at[:, 59:62].set(W1[:, 117:120]))

    full = lambda *shape: pl.BlockSpec(shape, lambda i: (0,) * len(shape))
    sig, rgb = pl.pallas_call(
        _tc_body,
        grid=(grid,),
        in_specs=[
            pl.BlockSpec((3, _BLK), lambda i: (0, i)),
            pl.BlockSpec((3, _BLK), lambda i: (0, i)),
            full(_N_GRID, 3),
            full(3, _NR, _N_GRID),
            full(3, _NR, _N_GRID),
            full(3, 2, _N_GRID),
            full(_CH // 2, _NR),
            full(_CH, _CH // 2),
            full(_CH, _CH // 2),
            full(_CH, _CH),
            full(3, _CH),
            full(_CH, 1),
            full(_CH, 1),
            full(3, 1),
        ],
        out_specs=[
            pl.BlockSpec((1, _BLK), lambda i: (0, i)),
            pl.BlockSpec((3, _BLK), lambda i: (0, i)),
        ],
        out_shape=[
            jax.ShapeDtypeStruct((1, npts), jnp.float32),
            jax.ShapeDtypeStruct((3, npts), jnp.float32),
        ],
    )(xyz_t, dirs_t, vox_t, tl, td, zv, bp, asin, acos,
      W2, W3, b1[:, None], b2[:, None], b3[:, None])
    return sig[0], rgb.T
